# baseline (speedup 1.0000x reference)
import functools

import jax
import jax.numpy as jnp
from jax import lax
from jax.experimental import pallas as pl
from jax.experimental.pallas import tpu as pltpu
from jax.experimental.pallas import tpu_sc as plsc

HIDDEN = 128
LANES = 16
VPR = HIDDEN // LANES
NC, NS = 2, 16
NW = NC * NS
CHUNK = 128
EPS = 1e-12


def _rsqrt(x):
    i = plsc.bitcast(x, jnp.int32)
    i = jnp.int32(0x5F3759DF) - lax.shift_right_arithmetic(i, 1)
    y = plsc.bitcast(i, jnp.float32)
    half = x * jnp.float32(0.5)
    for _ in range(2):
        y = y * (jnp.float32(1.5) - half * y * y)
    return y


def _ln_rows(wrows, prows, orows, gvec, bvec, n_rows):
    def one_row(i):
        xs = []
        s = None
        q = None
        for j in range(VPR):
            x = wrows[i, pl.ds(j * LANES, LANES)] + prows[i, pl.ds(j * LANES, LANES)]
            xs.append(x)
            s = x if s is None else s + x
            q = x * x if q is None else q + x * x
        def lane_total(v):
            pre = plsc.cumsum(v)
            suf = lax.rev(plsc.cumsum(lax.rev(v, (0,))), (0,))
            return pre + suf - v

        mean = lane_total(s) * jnp.float32(1.0 / HIDDEN)
        msq = lane_total(q) * jnp.float32(1.0 / HIDDEN)
        var = msq - mean * mean
        rstd = _rsqrt(var + jnp.float32(EPS))
        for j in range(VPR):
            orows[i, pl.ds(j * LANES, LANES)] = (xs[j] - mean) * rstd * gvec[j] + bvec[j]

    def body(r, carry):
        one_row(2 * r)
        one_row(2 * r + 1)
        return carry

    lax.fori_loop(0, n_rows // 2, body, None)


def _sc_kernel(ids_hbm, pos_hbm, wt_hbm, pt_hbm, g_hbm, b_hbm, out_hbm,
               idw_v, idp_v, pt_sh, wb, pb, ob, gb_v, sems, n_tokens):
    sid = lax.axis_index("s")
    wid = sid * NC + lax.axis_index("c")
    per_w = n_tokens // NW
    steps = per_w // CHUNK
    w_base = wid * per_w

    pltpu.sync_copy(g_hbm, gb_v.at[0])
    pltpu.sync_copy(b_hbm, gb_v.at[1])
    gvec = [gb_v[0, pl.ds(j * LANES, LANES)] for j in range(VPR)]
    bvec = [gb_v[1, pl.ds(j * LANES, LANES)] for j in range(VPR)]

    pltpu.sync_copy(ids_hbm.at[pl.ds(w_base, per_w)], idw_v)
    pltpu.sync_copy(pos_hbm.at[pl.ds(w_base, per_w)], idp_v)

    @pl.when(sid == 0)
    def _():
        pltpu.sync_copy(pt_hbm, pt_sh)

    plsc.subcore_barrier()

    sem_w = (sems[0], sems[1])
    sem_p = (sems[2], sems[3])
    sem_o = (sems[4], sems[5])

    def gather(g, p):
        idx_w = idw_v.at[pl.ds(g * CHUNK, CHUNK)]
        idx_p = idp_v.at[pl.ds(g * CHUNK, CHUNK)]
        pltpu.make_async_copy(wt_hbm.at[idx_w], wb[p], sem_w[p]).start()
        pltpu.make_async_copy(pt_sh.at[idx_p], pb[p], sem_p[p]).start()

    def gather_wait(g, p):
        idx_w = idw_v.at[pl.ds(g * CHUNK, CHUNK)]
        idx_p = idp_v.at[pl.ds(g * CHUNK, CHUNK)]
        pltpu.make_async_copy(wt_hbm.at[idx_w], wb[p], sem_w[p]).wait()
        pltpu.make_async_copy(pt_sh.at[idx_p], pb[p], sem_p[p]).wait()

    def write(g, p):
        dst = out_hbm.at[pl.ds(w_base + g * CHUNK, CHUNK)]
        pltpu.make_async_copy(ob[p], dst, sem_o[p]).start()

    def write_wait(g, p):
        dst = out_hbm.at[pl.ds(w_base + g * CHUNK, CHUNK)]
        pltpu.make_async_copy(ob[p], dst, sem_o[p]).wait()

    def handle(g, p):
        @pl.when(g + 1 < steps)
        def _():
            gather(g + 1, 1 - p)

        gather_wait(g, p)

        @pl.when(g >= 2)
        def _():
            write_wait(g - 2, p)

        _ln_rows(wb[p], pb[p], ob[p], gvec, bvec, CHUNK)
        write(g, p)

    gather(0, 0)

    def pair(g2, carry):
        g = 2 * g2
        handle(g, 0)
        handle(g + 1, 1)
        return carry

    lax.fori_loop(0, steps // 2, pair, None)
    write_wait(steps - 2, 0)
    write_wait(steps - 1, 1)


def kernel(input_ids, position_ids, text_attn_masks, word_table, pos_table,
           ln_gamma, ln_beta):
    B, L = input_ids.shape
    n = B * L
    ids = jnp.transpose(input_ids).reshape(n).astype(jnp.int32)
    pos = jnp.transpose(position_ids).reshape(n).astype(jnp.int32)
    per_w = n // NW

    mesh = plsc.VectorSubcoreMesh(core_axis_name="c", subcore_axis_name="s")
    run = pl.kernel(
        functools.partial(_sc_kernel, n_tokens=n),
        out_type=jax.ShapeDtypeStruct((n, HIDDEN), jnp.float32),
        mesh=mesh,
        compiler_params=pltpu.CompilerParams(needs_layout_passes=False),
        scratch_types=[
            pltpu.VMEM((per_w,), jnp.int32),
            pltpu.VMEM((per_w,), jnp.int32),
            pltpu.VMEM_SHARED(pos_table.shape, jnp.float32),
            [pltpu.VMEM((CHUNK, HIDDEN), jnp.float32)] * 2,
            [pltpu.VMEM((CHUNK, HIDDEN), jnp.float32)] * 2,
            [pltpu.VMEM((CHUNK, HIDDEN), jnp.float32)] * 2,
            pltpu.VMEM((2, HIDDEN), jnp.float32),
            [pltpu.SemaphoreType.DMA] * 6,
        ],
    )
    out = run(ids, pos, word_table, pos_table, ln_gamma, ln_beta)
    out = out.reshape(L, B, HIDDEN).transpose(1, 0, 2)
    return (out, text_attn_masks)


# speedup vs baseline: 1.0875x; 1.0875x over previous
import functools

import jax
import jax.numpy as jnp
from jax import lax
from jax.experimental import pallas as pl
from jax.experimental.pallas import tpu as pltpu
from jax.experimental.pallas import tpu_sc as plsc

HIDDEN = 128
LANES = 16
VPR = HIDDEN // LANES
NC, NS = 2, 16
NW = NC * NS
CHUNK = 128
EPS = 1e-12


def _rsqrt(x):
    i = plsc.bitcast(x, jnp.int32)
    i = jnp.int32(0x5F3759DF) - lax.shift_right_arithmetic(i, 1)
    y = plsc.bitcast(i, jnp.float32)
    half = x * jnp.float32(0.5)
    for _ in range(2):
        y = y * (jnp.float32(1.5) - half * y * y)
    return y


def _ln_rows(wrows, prows, orows, gvec, bvec, n_rows):
    def one_row(i):
        xs = []
        s = None
        q = None
        for j in range(VPR):
            x = wrows[i, pl.ds(j * LANES, LANES)] + prows[i, pl.ds(j * LANES, LANES)]
            xs.append(x)
            s = x if s is None else s + x
            q = x * x if q is None else q + x * x
        mean = jnp.full((LANES,), jnp.sum(s) * jnp.float32(1.0 / HIDDEN))
        msq = jnp.full((LANES,), jnp.sum(q) * jnp.float32(1.0 / HIDDEN))
        var = msq - mean * mean
        rstd = _rsqrt(var + jnp.float32(EPS))
        for j in range(VPR):
            orows[i, pl.ds(j * LANES, LANES)] = (xs[j] - mean) * rstd * gvec[j] + bvec[j]

    def body(r, carry):
        one_row(2 * r)
        one_row(2 * r + 1)
        return carry

    lax.fori_loop(0, n_rows // 2, body, None)


def _sc_kernel(ids_hbm, pos_hbm, wt_hbm, pt_hbm, g_hbm, b_hbm, out_hbm,
               idw_v, idp_v, pt_sh, wb, pb, ob, gb_v, sems, n_tokens):
    sid = lax.axis_index("s")
    wid = sid * NC + lax.axis_index("c")

    pltpu.sync_copy(g_hbm, gb_v.at[0])
    pltpu.sync_copy(b_hbm, gb_v.at[1])
    gvec = [gb_v[0, pl.ds(j * LANES, LANES)] for j in range(VPR)]
    bvec = [gb_v[1, pl.ds(j * LANES, LANES)] for j in range(VPR)]

    steps = ids_hbm.shape[0]
    batch = ids_hbm.shape[1]
    pltpu.sync_copy(ids_hbm.at[:, pl.ds(wid * CHUNK, CHUNK)], idw_v)
    pltpu.sync_copy(pos_hbm.at[:, pl.ds(wid * CHUNK, CHUNK)], idp_v)

    @pl.when(sid == 0)
    def _():
        pltpu.sync_copy(pt_hbm, pt_sh)

    plsc.subcore_barrier()

    sem_w = (sems[0], sems[1])
    sem_p = (sems[2], sems[3])
    sem_o = (sems[4], sems[5])

    def gather(g, p):
        pltpu.make_async_copy(wt_hbm.at[idw_v.at[g]], wb[p], sem_w[p]).start()
        pltpu.make_async_copy(pt_sh.at[idp_v.at[g]], pb[p], sem_p[p]).start()

    def gather_wait(g, p):
        pltpu.make_async_copy(wt_hbm.at[idw_v.at[g]], wb[p], sem_w[p]).wait()
        pltpu.make_async_copy(pt_sh.at[idp_v.at[g]], pb[p], sem_p[p]).wait()

    def write(g, p):
        dst = out_hbm.at[pl.ds(g * batch + wid * CHUNK, CHUNK)]
        pltpu.make_async_copy(ob[p], dst, sem_o[p]).start()

    def write_wait(g, p):
        dst = out_hbm.at[pl.ds(g * batch + wid * CHUNK, CHUNK)]
        pltpu.make_async_copy(ob[p], dst, sem_o[p]).wait()

    def handle(g, p):
        @pl.when(g + 1 < steps)
        def _():
            gather(g + 1, 1 - p)

        gather_wait(g, p)

        @pl.when(g >= 2)
        def _():
            write_wait(g - 2, p)

        _ln_rows(wb[p], pb[p], ob[p], gvec, bvec, CHUNK)
        write(g, p)

    gather(0, 0)

    def pair(g2, carry):
        g = 2 * g2
        handle(g, 0)
        handle(g + 1, 1)
        return carry

    lax.fori_loop(0, steps // 2, pair, None)
    write_wait(steps - 2, 0)
    write_wait(steps - 1, 1)


def kernel(input_ids, position_ids, text_attn_masks, word_table, pos_table,
           ln_gamma, ln_beta):
    B, L = input_ids.shape
    n = B * L
    ids = jnp.transpose(input_ids).astype(jnp.int32)
    pos = jnp.transpose(position_ids).astype(jnp.int32)

    mesh = plsc.VectorSubcoreMesh(core_axis_name="c", subcore_axis_name="s")
    run = pl.kernel(
        functools.partial(_sc_kernel, n_tokens=n),
        out_type=jax.ShapeDtypeStruct((n, HIDDEN), jnp.float32),
        mesh=mesh,
        compiler_params=pltpu.CompilerParams(needs_layout_passes=False),
        scratch_types=[
            pltpu.VMEM((L, CHUNK), jnp.int32),
            pltpu.VMEM((L, CHUNK), jnp.int32),
            pltpu.VMEM_SHARED(pos_table.shape, jnp.float32),
            [pltpu.VMEM((CHUNK, HIDDEN), jnp.float32)] * 2,
            [pltpu.VMEM((CHUNK, HIDDEN), jnp.float32)] * 2,
            [pltpu.VMEM((CHUNK, HIDDEN), jnp.float32)] * 2,
            pltpu.VMEM((2, HIDDEN), jnp.float32),
            [pltpu.SemaphoreType.DMA] * 6,
        ],
    )
    out = run(ids, pos, word_table, pos_table, ln_gamma, ln_beta)
    out = out.reshape(L, B, HIDDEN).transpose(1, 0, 2)
    return (out, text_attn_masks)


# speedup vs baseline: 1.1020x; 1.0133x over previous
import functools

import jax
import jax.numpy as jnp
from jax import lax
from jax.experimental import pallas as pl
from jax.experimental.pallas import tpu as pltpu
from jax.experimental.pallas import tpu_sc as plsc

HIDDEN = 128
LANES = 16
VPR = HIDDEN // LANES
NC, NS = 2, 16
NW = NC * NS
CHUNK = 128
EPS = 1e-12


def _rsqrt(x):
    i = plsc.bitcast(x, jnp.int32)
    i = jnp.int32(0x5F3759DF) - lax.shift_right_arithmetic(i, 1)
    y = plsc.bitcast(i, jnp.float32)
    half = x * jnp.float32(0.5)
    y = y * (jnp.float32(1.5) - half * y * y)
    return y


def _ln_rows(wrows, prows, orows, gvec, bvec, n_rows):
    def one_row(i):
        xs = []
        s = None
        q = None
        for j in range(VPR):
            x = wrows[i, pl.ds(j * LANES, LANES)] + prows[i, pl.ds(j * LANES, LANES)]
            xs.append(x)
            s = x if s is None else s + x
            q = x * x if q is None else q + x * x
        mean = jnp.full((LANES,), jnp.sum(s) * jnp.float32(1.0 / HIDDEN))
        msq = jnp.full((LANES,), jnp.sum(q) * jnp.float32(1.0 / HIDDEN))
        var = msq - mean * mean
        rstd = _rsqrt(var + jnp.float32(EPS))
        for j in range(VPR):
            orows[i, pl.ds(j * LANES, LANES)] = (xs[j] - mean) * rstd * gvec[j] + bvec[j]

    def body(r, carry):
        one_row(2 * r)
        one_row(2 * r + 1)
        return carry

    lax.fori_loop(0, n_rows // 2, body, None)


def _sc_kernel(ids_hbm, pos_hbm, wt_hbm, pt_hbm, g_hbm, b_hbm, out_hbm,
               idw_v, idp_v, pt_sh, wb, pb, ob, gb_v, sems, n_tokens):
    sid = lax.axis_index("s")
    wid = sid * NC + lax.axis_index("c")

    pltpu.sync_copy(g_hbm, gb_v.at[0])
    pltpu.sync_copy(b_hbm, gb_v.at[1])
    gvec = [gb_v[0, pl.ds(j * LANES, LANES)] for j in range(VPR)]
    bvec = [gb_v[1, pl.ds(j * LANES, LANES)] for j in range(VPR)]

    steps = ids_hbm.shape[0]
    batch = ids_hbm.shape[1]
    pltpu.sync_copy(ids_hbm.at[:, pl.ds(wid * CHUNK, CHUNK)], idw_v)
    pltpu.sync_copy(pos_hbm.at[:, pl.ds(wid * CHUNK, CHUNK)], idp_v)

    @pl.when(sid == 0)
    def _():
        pltpu.sync_copy(pt_hbm, pt_sh)

    plsc.subcore_barrier()

    sem_w = (sems[0], sems[1])
    sem_p = (sems[2], sems[3])
    sem_o = (sems[4], sems[5])

    def gather(g, p):
        pltpu.make_async_copy(wt_hbm.at[idw_v.at[g]], wb[p], sem_w[p]).start()
        pltpu.make_async_copy(pt_sh.at[idp_v.at[g]], pb[p], sem_p[p]).start()

    def gather_wait(g, p):
        pltpu.make_async_copy(wt_hbm.at[idw_v.at[g]], wb[p], sem_w[p]).wait()
        pltpu.make_async_copy(pt_sh.at[idp_v.at[g]], pb[p], sem_p[p]).wait()

    def write(g, p):
        dst = out_hbm.at[pl.ds(g * batch + wid * CHUNK, CHUNK)]
        pltpu.make_async_copy(ob[p], dst, sem_o[p]).start()

    def write_wait(g, p):
        dst = out_hbm.at[pl.ds(g * batch + wid * CHUNK, CHUNK)]
        pltpu.make_async_copy(ob[p], dst, sem_o[p]).wait()

    def handle(g, p):
        @pl.when(g + 1 < steps)
        def _():
            gather(g + 1, 1 - p)

        gather_wait(g, p)

        @pl.when(g >= 2)
        def _():
            write_wait(g - 2, p)

        _ln_rows(wb[p], pb[p], ob[p], gvec, bvec, CHUNK)
        write(g, p)

    gather(0, 0)

    def pair(g2, carry):
        g = 2 * g2
        handle(g, 0)
        handle(g + 1, 1)
        return carry

    lax.fori_loop(0, steps // 2, pair, None)
    write_wait(steps - 2, 0)
    write_wait(steps - 1, 1)


def kernel(input_ids, position_ids, text_attn_masks, word_table, pos_table,
           ln_gamma, ln_beta):
    B, L = input_ids.shape
    n = B * L
    ids = jnp.transpose(input_ids).astype(jnp.int32)
    pos = jnp.transpose(position_ids).astype(jnp.int32)

    mesh = plsc.VectorSubcoreMesh(core_axis_name="c", subcore_axis_name="s")
    run = pl.kernel(
        functools.partial(_sc_kernel, n_tokens=n),
        out_type=jax.ShapeDtypeStruct((n, HIDDEN), jnp.float32),
        mesh=mesh,
        compiler_params=pltpu.CompilerParams(needs_layout_passes=False),
        scratch_types=[
            pltpu.VMEM((L, CHUNK), jnp.int32),
            pltpu.VMEM((L, CHUNK), jnp.int32),
            pltpu.VMEM_SHARED(pos_table.shape, jnp.float32),
            [pltpu.VMEM((CHUNK, HIDDEN), jnp.float32)] * 2,
            [pltpu.VMEM((CHUNK, HIDDEN), jnp.float32)] * 2,
            [pltpu.VMEM((CHUNK, HIDDEN), jnp.float32)] * 2,
            pltpu.VMEM((2, HIDDEN), jnp.float32),
            [pltpu.SemaphoreType.DMA] * 6,
        ],
    )
    out = run(ids, pos, word_table, pos_table, ln_gamma, ln_beta)
    out = out.reshape(L, B, HIDDEN).transpose(1, 0, 2)
    return (out, text_attn_masks)


# speedup vs baseline: 1.1023x; 1.0003x over previous
import jax
import jax.numpy as jnp
from jax import lax
from jax.experimental import pallas as pl
from jax.experimental.pallas import tpu as pltpu
from jax.experimental.pallas import tpu_sc as plsc

HIDDEN = 128
LANES = 16
VPR = HIDDEN // LANES
NC, NS = 2, 16
NW = NC * NS
CHUNK = 128
EPS = 1e-12


def _rsqrt(x):
    i = plsc.bitcast(x, jnp.int32)
    i = jnp.int32(0x5F3759DF) - lax.shift_right_arithmetic(i, 1)
    y = plsc.bitcast(i, jnp.float32)
    half = x * jnp.float32(0.5)
    y = y * (jnp.float32(1.5) - half * y * y)
    return y


def _ln_rows(wrows, prows, orows, gvec, bvec, n_rows):
    def one_row(i):
        xs = []
        s = None
        q = None
        for j in range(VPR):
            x = wrows[i, pl.ds(j * LANES, LANES)] + prows[i, pl.ds(j * LANES, LANES)]
            xs.append(x)
            s = x if s is None else s + x
            q = x * x if q is None else q + x * x
        mean = jnp.full((LANES,), jnp.sum(s) * jnp.float32(1.0 / HIDDEN))
        msq = jnp.full((LANES,), jnp.sum(q) * jnp.float32(1.0 / HIDDEN))
        var = msq - mean * mean
        rstd = _rsqrt(var + jnp.float32(EPS))
        for j in range(VPR):
            orows[i, pl.ds(j * LANES, LANES)] = (xs[j] - mean) * rstd * gvec[j] + bvec[j]

    def body(r, carry):
        one_row(2 * r)
        one_row(2 * r + 1)
        return carry

    lax.fori_loop(0, n_rows // 2, body, None)


def _sc_kernel(ids_hbm, pos_hbm, wt_hbm, pt_hbm, g_hbm, b_hbm, out_hbm,
               idw_v, idp_v, pt_sh, wb, pb, ob, gb_v, sems):
    sid = lax.axis_index("s")
    wid = sid * NC + lax.axis_index("c")

    pltpu.sync_copy(g_hbm, gb_v.at[0])
    pltpu.sync_copy(b_hbm, gb_v.at[1])
    gvec = [gb_v[0, pl.ds(j * LANES, LANES)] for j in range(VPR)]
    bvec = [gb_v[1, pl.ds(j * LANES, LANES)] for j in range(VPR)]

    steps = ids_hbm.shape[0]
    batch = ids_hbm.shape[1]
    pltpu.sync_copy(ids_hbm.at[:, pl.ds(wid * CHUNK, CHUNK)], idw_v)
    pltpu.sync_copy(pos_hbm.at[:, pl.ds(wid * CHUNK, CHUNK)], idp_v)

    @pl.when(sid == 0)
    def _():
        pltpu.sync_copy(pt_hbm, pt_sh)

    plsc.subcore_barrier()

    sem_w = (sems[0], sems[1])
    sem_p = (sems[2], sems[3])
    sem_o = (sems[4], sems[5])

    def gather(g, p):
        pltpu.make_async_copy(wt_hbm.at[idw_v.at[g]], wb[p], sem_w[p]).start()
        pltpu.make_async_copy(pt_sh.at[idp_v.at[g]], pb[p], sem_p[p]).start()

    def gather_wait(g, p):
        pltpu.make_async_copy(wt_hbm.at[idw_v.at[g]], wb[p], sem_w[p]).wait()
        pltpu.make_async_copy(pt_sh.at[idp_v.at[g]], pb[p], sem_p[p]).wait()

    def write(g, p):
        dst = out_hbm.at[pl.ds(g * batch + wid * CHUNK, CHUNK)]
        pltpu.make_async_copy(ob[p], dst, sem_o[p]).start()

    def write_wait(g, p):
        dst = out_hbm.at[pl.ds(g * batch + wid * CHUNK, CHUNK)]
        pltpu.make_async_copy(ob[p], dst, sem_o[p]).wait()

    def handle(g, p):
        @pl.when(g + 1 < steps)
        def _():
            gather(g + 1, 1 - p)

        gather_wait(g, p)

        @pl.when(g >= 2)
        def _():
            write_wait(g - 2, p)

        _ln_rows(wb[p], pb[p], ob[p], gvec, bvec, CHUNK)
        write(g, p)

    gather(0, 0)

    def pair(g2, carry):
        g = 2 * g2
        handle(g, 0)
        handle(g + 1, 1)
        return carry

    lax.fori_loop(0, steps // 2, pair, None)
    write_wait(steps - 2, 0)
    write_wait(steps - 1, 1)


def kernel(input_ids, position_ids, text_attn_masks, word_table, pos_table,
           ln_gamma, ln_beta):
    B, L = input_ids.shape
    n = B * L
    ids = jnp.transpose(input_ids).astype(jnp.int32)
    pos = jnp.transpose(position_ids).astype(jnp.int32)

    mesh = plsc.VectorSubcoreMesh(core_axis_name="c", subcore_axis_name="s")
    run = pl.kernel(
        _sc_kernel,
        out_type=jax.ShapeDtypeStruct((n, HIDDEN), jnp.float32),
        mesh=mesh,
        compiler_params=pltpu.CompilerParams(needs_layout_passes=False),
        scratch_types=[
            pltpu.VMEM((L, CHUNK), jnp.int32),
            pltpu.VMEM((L, CHUNK), jnp.int32),
            pltpu.VMEM_SHARED(pos_table.shape, jnp.float32),
            [pltpu.VMEM((CHUNK, HIDDEN), jnp.float32)] * 2,
            [pltpu.VMEM((CHUNK, HIDDEN), jnp.float32)] * 2,
            [pltpu.VMEM((CHUNK, HIDDEN), jnp.float32)] * 2,
            pltpu.VMEM((2, HIDDEN), jnp.float32),
            [pltpu.SemaphoreType.DMA] * 6,
        ],
    )
    out = run(ids, pos, word_table, pos_table, ln_gamma, ln_beta)
    out = out.reshape(L, B, HIDDEN).transpose(1, 0, 2)
    return (out, text_attn_masks)
